# Initial kernel scaffold; baseline (speedup 1.0000x reference)
import jax
import jax.numpy as jnp
from jax.experimental import pallas as pl


def kernel(indices, table):
    raise NotImplementedError("write your pallas kernel here")


# speedup vs baseline: 2.9711x; 2.9711x over previous
import functools

import jax
import jax.numpy as jnp
from jax import lax
from jax.experimental import pallas as pl
from jax.experimental.pallas import tpu as pltpu
from jax.experimental.pallas import tpu_sc as plsc

_CHUNK = 128


@functools.lru_cache(maxsize=None)
def _build(N, V, D, NC, NS):
    NW = NC * NS
    n_per_w = N // NW
    n_chunks = n_per_w // _CHUNK
    mesh = plsc.VectorSubcoreMesh(core_axis_name="c", subcore_axis_name="s")

    @functools.partial(
        pl.kernel,
        mesh=mesh,
        out_type=jax.ShapeDtypeStruct((N, D), jnp.float32),
        scratch_types=[
            pltpu.VMEM((n_chunks, _CHUNK), jnp.int32),
            pltpu.VMEM((_CHUNK, D), jnp.float32),
            pltpu.SemaphoreType.DMA,
        ],
    )
    def k(idx_hbm, table_hbm, out_hbm, idx_v, rows_v, gsem):
        wid = lax.axis_index("s") * NC + lax.axis_index("c")
        chunk0 = wid * n_chunks
        pltpu.sync_copy(idx_hbm.at[wid], idx_v)

        def body(j, carry):
            pltpu.async_copy(table_hbm.at[idx_v.at[j]], rows_v, gsem).wait()
            base = (chunk0 + j) * _CHUNK
            pltpu.sync_copy(rows_v, out_hbm.at[pl.ds(base, _CHUNK)])
            return carry

        lax.fori_loop(0, n_chunks, body, 0)

    return k


def kernel(indices, table):
    B, L = indices.shape
    V, D = table.shape
    N = B * L
    info = plsc.get_sparse_core_info()
    NW = info.num_cores * info.num_subcores
    idx_flat = indices.reshape(NW, N // (NW * _CHUNK), _CHUNK).astype(jnp.int32)
    k = _build(N, V, D, info.num_cores, info.num_subcores)
    out = k(idx_flat, table)
    return out.reshape(B, L, D)


# speedup vs baseline: 3.1300x; 1.0535x over previous
import functools

import jax
import jax.numpy as jnp
from jax import lax
from jax.experimental import pallas as pl
from jax.experimental.pallas import tpu as pltpu
from jax.experimental.pallas import tpu_sc as plsc

_CHUNK = 128


@functools.lru_cache(maxsize=None)
def _build(N, V, D, NC, NS):
    NW = NC * NS
    n_per_w = N // NW
    n_chunks = n_per_w // _CHUNK
    assert n_chunks % 2 == 0 and n_chunks >= 4
    mesh = plsc.VectorSubcoreMesh(core_axis_name="c", subcore_axis_name="s")

    @functools.partial(
        pl.kernel,
        mesh=mesh,
        out_type=jax.ShapeDtypeStruct((N, D), jnp.float32),
        scratch_types=[
            pltpu.VMEM((n_chunks, _CHUNK), jnp.int32),
            pltpu.VMEM((2, _CHUNK, D), jnp.float32),
            pltpu.SemaphoreType.DMA,
            pltpu.SemaphoreType.DMA,
            pltpu.SemaphoreType.DMA,
            pltpu.SemaphoreType.DMA,
        ],
    )
    def k(idx_hbm, table_hbm, out_hbm, idx_v, rows_v, g0, g1, s0, s1):
        wid = lax.axis_index("s") * NC + lax.axis_index("c")
        chunk0 = wid * n_chunks
        gsem = (g0, g1)
        ssem = (s0, s1)
        pltpu.sync_copy(idx_hbm.at[wid], idx_v)

        def start_gather(j, b):
            pltpu.async_copy(table_hbm.at[idx_v.at[j]], rows_v.at[b], gsem[b])

        def wait_gather(b):
            pltpu.make_async_copy(
                table_hbm.at[pl.ds(0, _CHUNK)], rows_v.at[b], gsem[b]
            ).wait()

        def start_store(j, b):
            pltpu.async_copy(
                rows_v.at[b],
                out_hbm.at[pl.ds((chunk0 + j) * _CHUNK, _CHUNK)],
                ssem[b],
            )

        def wait_store(b):
            pltpu.make_async_copy(
                rows_v.at[b], out_hbm.at[pl.ds(0, _CHUNK)], ssem[b]
            ).wait()

        start_gather(0, 0)
        wait_gather(0)
        start_gather(1, 1)
        start_store(0, 0)

        def body(i, carry):
            for b_off in range(2):
                j = 2 * i + 1 + b_off
                b = 1 - b_off
                wait_gather(b)
                start_store(j, b)
                wait_store(1 - b)
                start_gather(j + 1, 1 - b)
            return carry

        lax.fori_loop(0, (n_chunks - 2) // 2, body, 0)

        wait_gather(1)
        start_store(n_chunks - 1, 1)
        wait_store(0)
        wait_store(1)

    return k


def kernel(indices, table):
    B, L = indices.shape
    V, D = table.shape
    N = B * L
    info = plsc.get_sparse_core_info()
    NW = info.num_cores * info.num_subcores
    idx_flat = indices.reshape(NW, N // (NW * _CHUNK), _CHUNK).astype(jnp.int32)
    k = _build(N, V, D, info.num_cores, info.num_subcores)
    out = k(idx_flat, table)
    return out.reshape(B, L, D)


# speedup vs baseline: 3.3251x; 1.0623x over previous
import functools

import jax
import jax.numpy as jnp
from jax import lax
from jax.experimental import pallas as pl
from jax.experimental.pallas import tpu as pltpu
from jax.experimental.pallas import tpu_sc as plsc

_CHUNK = 128


@functools.lru_cache(maxsize=None)
def _build(N, V, D, NC, NS):
    NW = NC * NS
    n_per_w = N // NW
    n_chunks = n_per_w // _CHUNK
    assert n_chunks == 50
    mesh = plsc.VectorSubcoreMesh(core_axis_name="c", subcore_axis_name="s")

    @functools.partial(
        pl.kernel,
        mesh=mesh,
        out_type=jax.ShapeDtypeStruct((N, D), jnp.float32),
        scratch_types=[
            pltpu.VMEM((n_chunks, _CHUNK), jnp.int32),
            pltpu.VMEM((4, _CHUNK, D), jnp.float32),
            pltpu.SemaphoreType.DMA,
            pltpu.SemaphoreType.DMA,
            pltpu.SemaphoreType.DMA,
            pltpu.SemaphoreType.DMA,
            pltpu.SemaphoreType.DMA,
            pltpu.SemaphoreType.DMA,
            pltpu.SemaphoreType.DMA,
            pltpu.SemaphoreType.DMA,
        ],
    )
    def k(idx_hbm, table_hbm, out_hbm, idx_v, rows_v,
          g0, g1, g2, g3, s0, s1, s2, s3):
        wid = lax.axis_index("s") * NC + lax.axis_index("c")
        chunk0 = wid * n_chunks
        gsem = (g0, g1, g2, g3)
        ssem = (s0, s1, s2, s3)
        pltpu.sync_copy(idx_hbm.at[wid], idx_v)

        def start_gather(j, b):
            pltpu.async_copy(table_hbm.at[idx_v.at[j]], rows_v.at[b], gsem[b])

        def wait_gather(b):
            pltpu.make_async_copy(
                table_hbm.at[pl.ds(0, _CHUNK)], rows_v.at[b], gsem[b]
            ).wait()

        def start_store(j, b):
            pltpu.async_copy(
                rows_v.at[b],
                out_hbm.at[pl.ds((chunk0 + j) * _CHUNK, _CHUNK)],
                ssem[b],
            )

        def wait_store(b):
            pltpu.make_async_copy(
                rows_v.at[b], out_hbm.at[pl.ds(0, _CHUNK)], ssem[b]
            ).wait()

        start_gather(0, 0)
        start_gather(1, 1)
        start_gather(2, 2)
        wait_gather(0)
        start_store(0, 0)
        start_gather(3, 3)
        wait_gather(1)
        start_store(1, 1)
        wait_store(0)
        start_gather(4, 0)
        wait_gather(2)
        start_store(2, 2)
        wait_store(1)
        start_gather(5, 1)
        wait_gather(3)
        start_store(3, 3)
        wait_store(2)
        start_gather(6, 2)

        def body(i, carry):
            for u in range(4):
                j = 4 + 4 * i + u
                b = u
                wait_gather(b)
                start_store(j, b)
                wait_store((u + 3) % 4)
                start_gather(j + 3, (u + 3) % 4)
            return carry

        lax.fori_loop(0, 10, body, 0)

        wait_gather(0)
        start_store(44, 0)
        wait_store(3)
        start_gather(47, 3)
        wait_gather(1)
        start_store(45, 1)
        wait_store(0)
        start_gather(48, 0)
        wait_gather(2)
        start_store(46, 2)
        wait_store(1)
        start_gather(49, 1)
        wait_gather(3)
        start_store(47, 3)
        wait_store(2)
        wait_gather(0)
        start_store(48, 0)
        wait_store(3)
        wait_gather(1)
        start_store(49, 1)
        wait_store(0)
        wait_store(1)

    return k


def kernel(indices, table):
    B, L = indices.shape
    V, D = table.shape
    N = B * L
    info = plsc.get_sparse_core_info()
    NW = info.num_cores * info.num_subcores
    idx_flat = indices.reshape(NW, N // (NW * _CHUNK), _CHUNK).astype(jnp.int32)
    k = _build(N, V, D, info.num_cores, info.num_subcores)
    out = k(idx_flat, table)
    return out.reshape(B, L, D)


# speedup vs baseline: 5.9535x; 1.7905x over previous
import functools

import jax
import jax.numpy as jnp
from jax import lax
from jax.experimental import pallas as pl
from jax.experimental.pallas import tpu as pltpu
from jax.experimental.pallas import tpu_sc as plsc

_NB = 2


@functools.lru_cache(maxsize=None)
def _build(B, L, V, D, NC, NS):
    NW = NC * NS
    n_per_w = B * L // NW
    b_per_w = B // NW
    chunk = _NB * L
    n_chunks = n_per_w // chunk
    assert chunk <= 128 and n_chunks % 4 == 0 and n_chunks >= 8
    mesh = plsc.VectorSubcoreMesh(core_axis_name="c", subcore_axis_name="s")

    @functools.partial(
        pl.kernel,
        mesh=mesh,
        out_type=jax.ShapeDtypeStruct((B, L, D), jnp.float32),
        scratch_types=[
            pltpu.VMEM((n_chunks, _NB * L), jnp.int32),
            pltpu.VMEM((4, _NB * L, D), jnp.float32),
            pltpu.SemaphoreType.DMA,
            pltpu.SemaphoreType.DMA,
            pltpu.SemaphoreType.DMA,
            pltpu.SemaphoreType.DMA,
            pltpu.SemaphoreType.DMA,
            pltpu.SemaphoreType.DMA,
            pltpu.SemaphoreType.DMA,
            pltpu.SemaphoreType.DMA,
        ],
    )
    def k(idx_hbm, table_hbm, out_hbm, idx_v, rows_v,
          g0, g1, g2, g3, s0, s1, s2, s3):
        wid = lax.axis_index("s") * NC + lax.axis_index("c")
        batch0 = wid * b_per_w
        gsem = (g0, g1, g2, g3)
        ssem = (s0, s1, s2, s3)
        pltpu.sync_copy(idx_hbm.at[wid], idx_v)

        def start_gather(j, b):
            pltpu.async_copy(
                table_hbm.at[idx_v.at[j]],
                rows_v.at[b],
                gsem[b],
            )

        def wait_gather(j, b):
            pltpu.make_async_copy(
                table_hbm.at[idx_v.at[j]], rows_v.at[b], gsem[b]
            ).wait()

        def start_store(j, b):
            for u in range(_NB):
                pltpu.async_copy(
                    rows_v.at[b, pl.ds(u * L, L)],
                    out_hbm.at[batch0 + j * _NB + u],
                    ssem[b],
                )

        def wait_store(b):
            for u in range(_NB):
                pltpu.make_async_copy(
                    rows_v.at[b, pl.ds(0, L)], out_hbm.at[0], ssem[b]
                ).wait()

        start_gather(0, 0)
        start_gather(1, 1)
        start_gather(2, 2)
        wait_gather(0, 0)
        start_store(0, 0)
        start_gather(3, 3)

        def body(i, carry):
            for u in range(4):
                j = 1 + 4 * i + u
                b = (1 + u) % 4
                wait_gather(j, b)
                start_store(j, b)
                wait_store((b + 3) % 4)
                start_gather(j + 3, (b + 3) % 4)
            return carry

        lax.fori_loop(0, (n_chunks - 4) // 4, body, 0)

        for j in (n_chunks - 3, n_chunks - 2, n_chunks - 1):
            b = j % 4
            wait_gather(j, b)
            start_store(j, b)
            wait_store((b + 3) % 4)
        wait_store((n_chunks - 1) % 4)

    return k


def kernel(indices, table):
    B, L = indices.shape
    V, D = table.shape
    info = plsc.get_sparse_core_info()
    NW = info.num_cores * info.num_subcores
    n_chunks = B // (NW * _NB)
    idx_flat = indices.reshape(NW, n_chunks, _NB * L).astype(jnp.int32)
    k = _build(B, L, V, D, info.num_cores, info.num_subcores)
    return k(idx_flat, table)
